# baseline (speedup 1.0000x reference)
import math

import jax
import jax.numpy as jnp
from jax.experimental import pallas as pl
from jax.experimental.pallas import tpu as pltpu

_TEMP = 0.05
_BN = 512
_LN2 = math.log(2.0)
_SCALE = math.log2(math.e) / _TEMP


def _ce_kernel(x_ref, fa_ref, fb_ref, t_ref, out_ref, xb_ref, c_ref, s_ref,
               tacc_ref):
    j = pl.program_id(0)
    nj = pl.num_programs(0)
    bn = fa_ref.shape[0]

    @pl.when(j == 0)
    def _init():
        x = x_ref[...]
        norm2 = jnp.sum(x * x, axis=1, keepdims=True)
        c_ref[...] = _SCALE * jax.lax.rsqrt(norm2)
        xb_ref[...] = x.astype(jnp.bfloat16)
        s_ref[...] = jnp.zeros_like(s_ref)
        tacc_ref[...] = jnp.zeros_like(tacc_ref)

    xb = xb_ref[...]
    c = c_ref[...]
    t = t_ref[...]

    def _tile(f_ref, blk):
        y = jax.lax.dot_general(
            xb, f_ref[...], (((1,), (1,)), ((), ())),
            preferred_element_type=jnp.float32,
        )
        e = jnp.exp2(y * c)
        s_ref[...] += jnp.sum(e, axis=1, keepdims=True)
        cols = blk * bn + jax.lax.broadcasted_iota(jnp.int32, e.shape, 1)
        masked = jnp.where(cols == t, e, 0.0)
        tacc_ref[...] += jnp.sum(masked, axis=1, keepdims=True)

    _tile(fa_ref, j)
    _tile(fb_ref, nj + j)

    @pl.when(j == nj - 1)
    def _fin():
        per_row = (jnp.log2(s_ref[...]) - jnp.log2(tacc_ref[...])) * _LN2
        out_ref[...] = jnp.sum(per_row, keepdims=True) * (1.0 / per_row.shape[0])


def kernel(epoch, inputs, ema_inputs, part_out, score, targets, features,
           part_features):
    m, k = inputs.shape
    n = features.shape[0]
    nj = n // (2 * _BN)
    out = pl.pallas_call(
        _ce_kernel,
        grid=(nj,),
        in_specs=[
            pl.BlockSpec((m, k), lambda j: (0, 0)),
            pl.BlockSpec((_BN, k), lambda j: (j, 0)),
            pl.BlockSpec((_BN, k), lambda j: (nj + j, 0)),
            pl.BlockSpec((m, 1), lambda j: (0, 0)),
        ],
        out_specs=pl.BlockSpec((1, 1), lambda j: (0, 0)),
        out_shape=jax.ShapeDtypeStruct((1, 1), jnp.float32),
        scratch_shapes=[
            pltpu.VMEM((m, k), jnp.bfloat16),
            pltpu.VMEM((m, 1), jnp.float32),
            pltpu.VMEM((m, 1), jnp.float32),
            pltpu.VMEM((m, 1), jnp.float32),
        ],
    )(inputs, features, features, targets.reshape(m, 1))
    return out[0, 0]


# speedup vs baseline: 1.0146x; 1.0146x over previous
import math

import jax
import jax.numpy as jnp
from jax.experimental import pallas as pl
from jax.experimental.pallas import tpu as pltpu

_TEMP = 0.05
_BN = 512
_LN2 = math.log(2.0)
_SCALE = math.log2(math.e) / _TEMP


def _ce_kernel(x_ref, f_ref, t_ref, out_ref, c_ref, s_ref, tacc_ref):
    j = pl.program_id(0)
    nj = pl.num_programs(0)
    bn = f_ref.shape[0]

    @pl.when(j == 0)
    def _init():
        x = x_ref[...]
        norm2 = jnp.sum(x * x, axis=1, keepdims=True)
        c_ref[...] = _SCALE * jax.lax.rsqrt(norm2)
        s_ref[...] = jnp.zeros_like(s_ref)
        tacc_ref[...] = jnp.zeros_like(tacc_ref)

    y = jax.lax.dot_general(
        x_ref[...].astype(jnp.bfloat16), f_ref[...], (((1,), (1,)), ((), ())),
        preferred_element_type=jnp.float32,
    )
    logits = y * c_ref[...]
    s_ref[...] += jnp.sum(jnp.exp2(logits), axis=1, keepdims=True)
    cols = j * bn + jax.lax.broadcasted_iota(jnp.int32, logits.shape, 1)
    masked = jnp.where(cols == t_ref[...], logits, 0.0)
    tacc_ref[...] += jnp.sum(masked, axis=1, keepdims=True)

    @pl.when(j == nj - 1)
    def _fin():
        per_row = (jnp.log2(s_ref[...]) - tacc_ref[...]) * _LN2
        out_ref[...] = jnp.sum(per_row, keepdims=True) * (1.0 / per_row.shape[0])


def kernel(epoch, inputs, ema_inputs, part_out, score, targets, features,
           part_features):
    m, k = inputs.shape
    n = features.shape[0]
    out = pl.pallas_call(
        _ce_kernel,
        grid=(n // _BN,),
        in_specs=[
            pl.BlockSpec((m, k), lambda j: (0, 0)),
            pl.BlockSpec((_BN, k), lambda j: (j, 0)),
            pl.BlockSpec((m, 1), lambda j: (0, 0)),
        ],
        out_specs=pl.BlockSpec((1, 1), lambda j: (0, 0)),
        out_shape=jax.ShapeDtypeStruct((1, 1), jnp.float32),
        scratch_shapes=[
            pltpu.VMEM((m, 1), jnp.float32),
            pltpu.VMEM((m, 1), jnp.float32),
            pltpu.VMEM((m, 1), jnp.float32),
        ],
    )(inputs, features, targets.reshape(m, 1))
    return out[0, 0]


# speedup vs baseline: 1.0152x; 1.0006x over previous
import math

import jax
import jax.numpy as jnp
from jax.experimental import pallas as pl
from jax.experimental.pallas import tpu as pltpu

_TEMP = 0.05
_BN = 2048
_LN2 = math.log(2.0)
_SCALE = math.log2(math.e) / _TEMP


def _ce_kernel(x_ref, f_ref, t_ref, out_ref, c_ref, s_ref, tacc_ref):
    j = pl.program_id(0)
    nj = pl.num_programs(0)
    bn = f_ref.shape[0]

    @pl.when(j == 0)
    def _init():
        x = x_ref[...]
        norm2 = jnp.sum(x * x, axis=1, keepdims=True)
        c_ref[...] = _SCALE * jax.lax.rsqrt(norm2)
        s_ref[...] = jnp.zeros_like(s_ref)
        tacc_ref[...] = jnp.zeros_like(tacc_ref)

    y = jax.lax.dot_general(
        x_ref[...].astype(jnp.bfloat16), f_ref[...], (((1,), (1,)), ((), ())),
        preferred_element_type=jnp.float32,
    )
    logits = y * c_ref[...]
    s_ref[...] += jnp.sum(jnp.exp2(logits), axis=1, keepdims=True)
    cols = j * bn + jax.lax.broadcasted_iota(jnp.int32, logits.shape, 1)
    masked = jnp.where(cols == t_ref[...], logits, 0.0)
    tacc_ref[...] += jnp.sum(masked, axis=1, keepdims=True)

    @pl.when(j == nj - 1)
    def _fin():
        per_row = (jnp.log2(s_ref[...]) - tacc_ref[...]) * _LN2
        out_ref[...] = jnp.sum(per_row, keepdims=True) * (1.0 / per_row.shape[0])


def kernel(epoch, inputs, ema_inputs, part_out, score, targets, features,
           part_features):
    m, k = inputs.shape
    n = features.shape[0]
    out = pl.pallas_call(
        _ce_kernel,
        grid=(n // _BN,),
        in_specs=[
            pl.BlockSpec((m, k), lambda j: (0, 0)),
            pl.BlockSpec((_BN, k), lambda j: (j, 0)),
            pl.BlockSpec((m, 1), lambda j: (0, 0)),
        ],
        out_specs=pl.BlockSpec((1, 1), lambda j: (0, 0)),
        out_shape=jax.ShapeDtypeStruct((1, 1), jnp.float32),
        scratch_shapes=[
            pltpu.VMEM((m, 1), jnp.float32),
            pltpu.VMEM((m, 1), jnp.float32),
            pltpu.VMEM((m, 1), jnp.float32),
        ],
    )(inputs, features, targets.reshape(m, 1))
    return out[0, 0]


# speedup vs baseline: 1.0555x; 1.0397x over previous
import math

import jax
import jax.numpy as jnp
from jax.experimental import pallas as pl
from jax.experimental.pallas import tpu as pltpu

_TEMP = 0.05
_BN = 1024
_LN2 = math.log(2.0)
_SCALE = math.log2(math.e) / _TEMP


def _ce_kernel(x_ref, f_ref, t_ref, out_ref, c_ref, s_ref, tacc_ref):
    j = pl.program_id(0)
    nj = pl.num_programs(0)
    bn = f_ref.shape[0]

    @pl.when(j == 0)
    def _init():
        x = x_ref[...]
        norm2 = jnp.sum(x * x, axis=1, keepdims=True)
        c_ref[...] = _SCALE * jax.lax.rsqrt(norm2)
        s_ref[...] = jnp.zeros_like(s_ref)
        tacc_ref[...] = jnp.zeros_like(tacc_ref)

    y = jax.lax.dot_general(
        x_ref[...].astype(jnp.bfloat16), f_ref[...], (((1,), (1,)), ((), ())),
        preferred_element_type=jnp.float32,
    )
    logits = y * c_ref[...]
    s_ref[...] += jnp.sum(jnp.exp2(logits), axis=1, keepdims=True)
    cols = j * bn + jax.lax.broadcasted_iota(jnp.int32, logits.shape, 1)
    masked = jnp.where(cols == t_ref[...], logits, 0.0)
    tacc_ref[...] += jnp.sum(masked, axis=1, keepdims=True)

    @pl.when(j == nj - 1)
    def _fin():
        per_row = (jnp.log2(s_ref[...]) - tacc_ref[...]) * _LN2
        out_ref[...] = jnp.sum(per_row, keepdims=True) * (1.0 / per_row.shape[0])


def kernel(epoch, inputs, ema_inputs, part_out, score, targets, features,
           part_features):
    m, k = inputs.shape
    n = features.shape[0]
    out = pl.pallas_call(
        _ce_kernel,
        grid=(n // _BN,),
        in_specs=[
            pl.BlockSpec((m, k), lambda j: (0, 0)),
            pl.BlockSpec((_BN, k), lambda j: (j, 0)),
            pl.BlockSpec((m, 1), lambda j: (0, 0)),
        ],
        out_specs=pl.BlockSpec((1, 1), lambda j: (0, 0)),
        out_shape=jax.ShapeDtypeStruct((1, 1), jnp.float32),
        scratch_shapes=[
            pltpu.VMEM((m, 1), jnp.float32),
            pltpu.VMEM((m, 1), jnp.float32),
            pltpu.VMEM((m, 1), jnp.float32),
        ],
    )(inputs, features, targets.reshape(m, 1))
    return out[0, 0]
